# baseline (speedup 1.0000x reference)
import functools

import jax
import jax.numpy as jnp
from jax import lax
from jax.experimental import pallas as pl
from jax.experimental.pallas import tpu as pltpu
from jax.experimental.pallas import tpu_sc as plsc

NUM_CORES = 2
NUM_SUBCORES = 16
NUM_WORKERS = NUM_CORES * NUM_SUBCORES
CHUNK = 128
KUNROLL = 8
NSTAGE = 4
F_AHEAD = 2


def _make_sc_kernel(B, S, V, E):
    n = B * S
    num_chunks = n // CHUNK
    assert n % CHUNK == 0 and num_chunks % NUM_WORKERS == 0
    assert S % 8 == 0
    cpw = num_chunks // NUM_WORKERS
    assert cpw % KUNROLL == 0 and cpw >= 2 * KUNROLL

    mesh = plsc.VectorSubcoreMesh(core_axis_name="c", subcore_axis_name="s")

    @functools.partial(
        pl.kernel,
        mesh=mesh,
        out_type=jax.ShapeDtypeStruct((n, E), jnp.float32),
        compiler_params=pltpu.CompilerParams(use_tc_tiling_on_sc=False),
        scratch_types=[
            pltpu.VMEM((cpw, CHUNK), jnp.int32),
            pltpu.VMEM((CHUNK,), jnp.int32),
            pltpu.VMEM((KUNROLL, CHUNK, E), jnp.float32),
            pltpu.VMEM_SHARED((NUM_SUBCORES, NSTAGE, CHUNK, E), jnp.float32),
        ]
        + [pltpu.SemaphoreType.DMA] * (KUNROLL + 2 * NSTAGE),
    )
    def k(ids_hbm, tok_hbm, pos2_hbm, iota_hbm, out_hbm,
          idx_v, iota_v, gbuf, stage, *sems):
        sid = lax.axis_index("s")
        wid = sid * NUM_CORES + lax.axis_index("c")
        gsem = sems[:KUNROLL]
        fsem = sems[KUNROLL:KUNROLL + NSTAGE]
        osem = sems[KUNROLL + NSTAGE:]
        gv = [gbuf.at[b] for b in range(KUNROLL)]
        sv = [stage.at[sid, b] for b in range(NSTAGE)]

        pltpu.sync_copy(ids_hbm.at[pl.ds(wid * cpw, cpw)], idx_v)
        pltpu.sync_copy(iota_hbm, iota_v)

        def fill(t, ss):
            phase = lax.rem((wid * cpw + t) * CHUNK, S)
            return pos2_hbm.at[pl.ds(phase, CHUNK)], sv[ss], fsem[ss]

        def outcp(t, ss):
            base = (wid * cpw + t) * CHUNK
            return sv[ss], out_hbm.at[pl.ds(base, CHUNK)], osem[ss]

        def start_fill(t, ss, drain):
            if drain:
                pltpu.make_async_copy(*outcp(t - NSTAGE, ss)).wait()
            pltpu.async_copy(*fill(t, ss))

        for j in range(F_AHEAD):
            start_fill(j, j % NSTAGE, False)

        @pl.loop(0, cpw, step=KUNROLL)
        def _(i):
            gds = [
                pltpu.async_copy(tok_hbm.at[idx_v.at[i + b]], gv[b], gsem[b])
                for b in range(KUNROLL)
            ]
            for b in range(KUNROLL):
                t = i + b
                ss = b % NSTAGE

                fs = (b + F_AHEAD) % NSTAGE
                ft = t + F_AHEAD

                @pl.when((ft >= NSTAGE) & (ft < cpw))
                def _():
                    start_fill(ft, fs, True)

                @pl.when((ft >= F_AHEAD) & (ft < NSTAGE))
                def _():
                    start_fill(ft, fs, False)

                gds[b].wait()
                pltpu.make_async_copy(*fill(t, ss)).wait()
                pltpu.sync_copy(gv[b], sv[ss].at[iota_v], add=True)
                pltpu.async_copy(*outcp(t, ss))

        for u in range(cpw - NSTAGE, cpw):
            pltpu.make_async_copy(*outcp(u, u % NSTAGE)).wait()

    return k


def kernel(input_ids, token_table, position_table):
    B, S = input_ids.shape
    V, E = token_table.shape
    n = B * S
    ids = input_ids.astype(jnp.int32).reshape(n // CHUNK, CHUNK)
    iota = jnp.arange(CHUNK, dtype=jnp.int32)
    pos2 = jnp.concatenate(
        [position_table[:S], position_table[:CHUNK]], axis=0)
    sc = _make_sc_kernel(B, S, V, E)
    out = sc(ids, token_table, pos2, iota)
    return out.reshape(B, S, E)


# speedup vs baseline: 1.1837x; 1.1837x over previous
import functools

import jax
import jax.numpy as jnp
from jax import lax
from jax.experimental import pallas as pl
from jax.experimental.pallas import tpu as pltpu
from jax.experimental.pallas import tpu_sc as plsc

NUM_CORES = 2
NUM_SUBCORES = 16
NUM_WORKERS = NUM_CORES * NUM_SUBCORES
LANES = 16
CHUNK = 100
KUNROLL = 8


def _make_sc_kernel(B, S, V, E):
    per_seq = S // CHUNK
    num_chunks = B * per_seq
    assert S % CHUNK == 0 and E % LANES == 0
    assert num_chunks % (NUM_WORKERS * KUNROLL) == 0
    cpw = num_chunks // NUM_WORKERS

    mesh = plsc.VectorSubcoreMesh(core_axis_name="c", subcore_axis_name="s")

    @functools.partial(
        pl.kernel,
        mesh=mesh,
        out_type=jax.ShapeDtypeStruct((B, S, E), jnp.float32),
        compiler_params=pltpu.CompilerParams(use_tc_tiling_on_sc=False),
        scratch_types=[
            pltpu.VMEM((cpw, CHUNK), jnp.int32),
            pltpu.VMEM((S, E), jnp.float32),
            pltpu.VMEM((KUNROLL, CHUNK, E), jnp.float32),
        ]
        + [pltpu.SemaphoreType.DMA] * (2 * KUNROLL),
    )
    def k(ids_hbm, tok_hbm, pos_hbm, out_hbm, idx_v, pos_v, gbuf, *sems):
        sid = lax.axis_index("s")
        wid = sid * NUM_CORES + lax.axis_index("c")
        gsem = sems[:KUNROLL]
        osem = sems[KUNROLL:]
        gv = [gbuf.at[b] for b in range(KUNROLL)]

        pltpu.sync_copy(ids_hbm.at[pl.ds(wid * cpw, cpw)], idx_v)
        pltpu.sync_copy(pos_hbm.at[pl.ds(0, S)], pos_v)

        def outcp(t, b):
            g = wid * cpw + t
            seq = lax.div(g, per_seq)
            half = lax.rem(g, per_seq)
            dst = out_hbm.at[seq, pl.ds(half * CHUNK, CHUNK)]
            return gv[b], dst, osem[b]

        @pl.loop(0, cpw, step=KUNROLL)
        def _(i):
            for b in range(KUNROLL):

                @pl.when(i > 0)
                def _():
                    pltpu.make_async_copy(*outcp(i - KUNROLL + b, b)).wait()

            gds = [
                pltpu.async_copy(tok_hbm.at[idx_v.at[i + b]], gv[b], gsem[b])
                for b in range(KUNROLL)
            ]

            for b in range(KUNROLL):
                t = i + b
                gds[b].wait()
                poff = (b % per_seq) * CHUNK

                @pl.loop(0, CHUNK)
                def _(r):
                    for c in range(E // LANES):
                        sl = pl.ds(c * LANES, LANES)
                        plsc.addupdate(gv[b].at[r, sl], pos_v[poff + r, sl])

                pltpu.async_copy(*outcp(t, b))

        for u in range(cpw - KUNROLL, cpw):
            pltpu.make_async_copy(*outcp(u, u % KUNROLL)).wait()

    return k


def kernel(input_ids, token_table, position_table):
    B, S = input_ids.shape
    V, E = token_table.shape
    ids = input_ids.astype(jnp.int32).reshape(-1, CHUNK)
    sc = _make_sc_kernel(B, S, V, E)
    return sc(ids, token_table, position_table)
